# Initial kernel scaffold; baseline (speedup 1.0000x reference)
import jax
import jax.numpy as jnp
from jax.experimental import pallas as pl


def kernel(x, current_val):
    raise NotImplementedError("write your pallas kernel here")


# speedup vs baseline: 1.2640x; 1.2640x over previous
import jax
import jax.numpy as jnp
from jax.experimental import pallas as pl
from jax.experimental.pallas import tpu as pltpu

S = 2048
ROWS_PER_BLOCK = 512


def _body(cv_ref, x_ref, o_ref):
    blk = pl.program_id(0)
    R, C = x_ref.shape
    cv = cv_ref[0]
    g = blk * R + jax.lax.broadcasted_iota(jnp.int32, (R, C), 0)
    r = jax.lax.rem(g, S)
    i = jnp.minimum(r, S - 1 - r)
    val = jnp.clip((i.astype(jnp.float32) - 991.0 + 2048.0 * cv) * (1.0 / 32.0),
                   0.0, 1.0)
    c = jax.lax.broadcasted_iota(jnp.int32, (R, C), 1)
    left = i + jnp.where(r >= S // 2, 1, 0)
    cond = (c >= left) & (c <= S - 1 - i)
    o_ref[...] = x_ref[...] * jnp.where(cond, val, 1.0)


def kernel(x, current_val):
    B, H, Sr, Sc = x.shape
    x2 = x.reshape(B * H * Sr, Sc)
    n_rows = x2.shape[0]
    grid = (n_rows // ROWS_PER_BLOCK,)
    out = pl.pallas_call(
        _body,
        grid=grid,
        in_specs=[
            pl.BlockSpec(memory_space=pltpu.SMEM),
            pl.BlockSpec((ROWS_PER_BLOCK, Sc), lambda b: (b, 0)),
        ],
        out_specs=pl.BlockSpec((ROWS_PER_BLOCK, Sc), lambda b: (b, 0)),
        out_shape=jax.ShapeDtypeStruct((n_rows, Sc), x.dtype),
    )(current_val, x2)
    return out.reshape(B, H, Sr, Sc)
